# baseline (speedup 1.0000x reference)
import jax
import jax.numpy as jnp
from jax.experimental import pallas as pl
from jax.experimental.pallas import tpu as pltpu

N = 8192
P, D, T = 1024, 512, 512
H = 512
E = 16
TN = 1024


def _gate_kernel(xp_ref, xd_ref, xt_ref, w1_ref, b1_ref, a_ref, g_ref,
                 bb_ref, w2_ref, b2_ref, out_ref):
    dn = (((1,), (1,)), ((), ()))
    x = jnp.concatenate([xp_ref[...], xd_ref[...], xt_ref[...]], axis=1)
    h = jax.lax.dot_general(x, w1_ref[...], dn,
                            preferred_element_type=jnp.float32)
    h = h + b1_ref[...][None, :]
    a = a_ref[0]
    h = jnp.maximum(h, 0.0) + a * jnp.minimum(h, 0.0)
    mu = jnp.mean(h, axis=-1, keepdims=True)
    c = h - mu
    var = jnp.mean(c * c, axis=-1, keepdims=True)
    hn = c / jnp.sqrt(var + 1e-5) * g_ref[...][None, :] + bb_ref[...][None, :]
    logits = jax.lax.dot_general(hn, w2_ref[...], dn,
                                 preferred_element_type=jnp.float32)
    logits = logits + b2_ref[...][None, :]
    lt = logits.T
    iota = jax.lax.broadcasted_iota(jnp.int32, lt.shape, 0)
    m1 = jnp.max(lt, axis=0, keepdims=True)
    idx1 = jnp.min(jnp.where(lt == m1, iota, E), axis=0, keepdims=True)
    is1 = iota == idx1
    masked = jnp.where(is1, -jnp.inf, lt)
    m2 = jnp.max(masked, axis=0, keepdims=True)
    idx2 = jnp.min(jnp.where(masked == m2, iota, E), axis=0, keepdims=True)
    e2 = jnp.exp(m2 - m1)
    z = 1.0 + e2
    pt = jnp.where(is1, 1.0 / z, jnp.where(iota == idx2, e2 / z, 0.0))
    out_ref[...] = pt


def kernel(protein_raw, v_prior, trust_vector, W1, b1, prelu_a, ln_g, ln_b,
           W2, b2):
    grid = (N // TN,)
    full = lambda i: (0, 0)
    row = lambda i: (i, 0)
    vec = lambda i: (0,)
    return pl.pallas_call(
        _gate_kernel,
        grid=grid,
        in_specs=[
            pl.BlockSpec((TN, P), row),
            pl.BlockSpec((TN, D), row),
            pl.BlockSpec((TN, T), row),
            pl.BlockSpec((H, P + D + T), full),
            pl.BlockSpec((H,), vec),
            pl.BlockSpec(memory_space=pltpu.SMEM),
            pl.BlockSpec((H,), vec),
            pl.BlockSpec((H,), vec),
            pl.BlockSpec((E, H), full),
            pl.BlockSpec((E,), vec),
        ],
        out_specs=pl.BlockSpec((E, TN), lambda i: (0, i)),
        out_shape=jax.ShapeDtypeStruct((E, N), jnp.float32),
        compiler_params=pltpu.CompilerParams(dimension_semantics=("arbitrary",)),
    )(protein_raw, v_prior, trust_vector, W1, b1, prelu_a.reshape(1), ln_g,
      ln_b, W2, b2).T
